# baseline (speedup 1.0000x reference)
import functools

import jax
import jax.numpy as jnp
from jax import lax
from jax.experimental import pallas as pl
from jax.experimental.pallas import tpu as pltpu
from jax.experimental.pallas import tpu_sc as plsc

N = 10000
E = 320000
D = 128
NC = 2
NS = 16
NW = NC * NS
LANES = 16

NPAD = 10240
EP = 327680
ROWS_PER_TILE = 80
EDGES_PER_TILE = ROWS_PER_TILE * 128
NROWS = EP // 128

_mesh = plsc.VectorSubcoreMesh(core_axis_name="c", subcore_axis_name="s")
_sc_params = pltpu.CompilerParams(needs_layout_passes=False)


def _wid():
    return lax.axis_index("c") * NS + lax.axis_index("s")


def _sc_deg_body(src_hbm, dst_hbm, ew_hbm, out_hbm,
                 src_v, dst_v, ew_v, degs_v, degd_v):
    w = _wid()
    c = lax.axis_index("c")
    s = lax.axis_index("s")
    row0 = w * ROWS_PER_TILE
    e0 = w * EDGES_PER_TILE

    @pl.loop(0, NPAD // LANES)
    def _(k):
        z = jnp.zeros((LANES,), jnp.float32)
        degs_v[pl.ds(k * LANES, LANES)] = z
        degd_v[pl.ds(k * LANES, LANES)] = z

    pltpu.sync_copy(src_hbm.at[pl.ds(row0, ROWS_PER_TILE)], src_v)
    pltpu.sync_copy(dst_hbm.at[pl.ds(row0, ROWS_PER_TILE)], dst_v)
    pltpu.sync_copy(ew_hbm.at[pl.ds(e0, EDGES_PER_TILE)], ew_v)

    @pl.loop(0, ROWS_PER_TILE)
    def _(i):
        for j in range(128 // LANES):
            sl = pl.ds(j * LANES, LANES)
            ev = ew_v[pl.ds(i * 128 + j * LANES, LANES)]
            plsc.addupdate_scatter(degs_v, [src_v[i, sl]], ev)
            plsc.addupdate_scatter(degd_v, [dst_v[i, sl]], ev)

    pltpu.sync_copy(degs_v, out_hbm.at[c, s, 0])
    pltpu.sync_copy(degd_v, out_hbm.at[c, s, 1])


@jax.jit
def _sc_deg(src2d, dst2d, ew):
    return pl.kernel(
        _sc_deg_body,
        out_type=jax.ShapeDtypeStruct((NC, NS, 2, NPAD), jnp.float32),
        mesh=_mesh,
        scratch_types=[
            pltpu.VMEM((ROWS_PER_TILE, 128), jnp.int32),
            pltpu.VMEM((ROWS_PER_TILE, 128), jnp.int32),
            pltpu.VMEM((EDGES_PER_TILE,), jnp.float32),
            pltpu.VMEM((NPAD,), jnp.float32),
            pltpu.VMEM((NPAD,), jnp.float32),
        ],
        compiler_params=_sc_params,
    )(src2d, dst2d, ew)


def _sc_coef_body(inv_hbm, src_hbm, dst_hbm, ew_hbm, coef_hbm,
                  invs_v, invd_v, src_v, dst_v, ew_v, coef_v):
    w = _wid()
    row0 = w * ROWS_PER_TILE
    e0 = w * EDGES_PER_TILE

    pltpu.sync_copy(inv_hbm.at[0], invs_v)
    pltpu.sync_copy(inv_hbm.at[1], invd_v)
    pltpu.sync_copy(src_hbm.at[pl.ds(row0, ROWS_PER_TILE)], src_v)
    pltpu.sync_copy(dst_hbm.at[pl.ds(row0, ROWS_PER_TILE)], dst_v)
    pltpu.sync_copy(ew_hbm.at[pl.ds(e0, EDGES_PER_TILE)], ew_v)

    @pl.loop(0, ROWS_PER_TILE)
    def _(i):
        for j in range(128 // LANES):
            sl = pl.ds(j * LANES, LANES)
            fl = pl.ds(i * 128 + j * LANES, LANES)
            a = plsc.load_gather(invs_v, [src_v[i, sl]])
            b = plsc.load_gather(invd_v, [dst_v[i, sl]])
            coef_v[fl] = ew_v[fl] * a * b

    pltpu.sync_copy(coef_v, coef_hbm.at[pl.ds(e0, EDGES_PER_TILE)])


@jax.jit
def _sc_coef(inv, src2d, dst2d, ew):
    return pl.kernel(
        _sc_coef_body,
        out_type=jax.ShapeDtypeStruct((EP,), jnp.float32),
        mesh=_mesh,
        scratch_types=[
            pltpu.VMEM((NPAD,), jnp.float32),
            pltpu.VMEM((NPAD,), jnp.float32),
            pltpu.VMEM((ROWS_PER_TILE, 128), jnp.int32),
            pltpu.VMEM((ROWS_PER_TILE, 128), jnp.int32),
            pltpu.VMEM((EDGES_PER_TILE,), jnp.float32),
            pltpu.VMEM((EDGES_PER_TILE,), jnp.float32),
        ],
        compiler_params=_sc_params,
    )(inv, src2d, dst2d, ew)


CH = 8
GSPLIT = 2
NCHUNK = ROWS_PER_TILE // CH


RPT0 = 112
RPT1 = 48
NR0TOT = NS * RPT0


def _sc_gcn_body(h_hbm, src_hbm, dst_hbm, coef_hbm, out_hbm,
                 srcs_v, dsts_v, coefs_v, bufs, gsems, ssems, stgsems,
                 acc_sh):
    c = lax.axis_index("c")
    s = lax.axis_index("s")
    nz = NPAD // NS

    with jax.named_scope("zero_phase"):
        @pl.loop(0, 128)
        def _(i):
            for j in range(128 // LANES):
                bufs[0][i, pl.ds(j * LANES, LANES)] = jnp.zeros(
                    (LANES,), jnp.float32)

        for k in range(nz // 128):
            pltpu.sync_copy(bufs[0], acc_sh.at[pl.ds(s * nz + k * 128, 128)])
        plsc.subcore_barrier()

    def _run(rpt, row0):
        e0 = row0 * 128
        nchunk = rpt // CH

        def _stage(ch, p):
            pltpu.async_copy(src_hbm.at[pl.ds(row0 + ch * CH, CH)],
                             srcs_v[p], stgsems[p])
            pltpu.async_copy(dst_hbm.at[pl.ds(row0 + ch * CH, CH)],
                             dsts_v[p], stgsems[p])
            pltpu.async_copy(coef_hbm.at[pl.ds(e0 + ch * CH * 128, CH * 128)],
                             coefs_v[p], stgsems[p])

        def _wait_stage(ch, p):
            pltpu.make_async_copy(src_hbm.at[pl.ds(row0 + ch * CH, CH)],
                                  srcs_v[p], stgsems[p]).wait()
            pltpu.make_async_copy(dst_hbm.at[pl.ds(row0 + ch * CH, CH)],
                                  dsts_v[p], stgsems[p]).wait()
            pltpu.make_async_copy(
                coef_hbm.at[pl.ds(e0 + ch * CH * 128, CH * 128)],
                coefs_v[p], stgsems[p]).wait()

        def _gather(p, r, b):
            for h in range(GSPLIT):
                hs = pl.ds(h * (128 // GSPLIT), 128 // GSPLIT)
                pltpu.async_copy(h_hbm.at[srcs_v[p].at[r, hs]],
                                 bufs[b].at[hs], gsems[b])

        def _wait_gather(p, r, b):
            for h in range(GSPLIT):
                hs = pl.ds(h * (128 // GSPLIT), 128 // GSPLIT)
                pltpu.make_async_copy(h_hbm.at[srcs_v[p].at[r, hs]],
                                      bufs[b].at[hs], gsems[b]).wait()

        def _scatter(p, r, b):
            pltpu.async_copy(bufs[b], acc_sh.at[dsts_v[p].at[r]], ssems[b],
                             add=True)

        def _wait_scatter(p, r, b):
            pltpu.make_async_copy(bufs[b], acc_sh.at[dsts_v[p].at[r]],
                                  ssems[b]).wait()

        _stage(0, 0)

        @pl.loop(0, nchunk // 2)
        def _(g):
            for p in range(2):
                ch = g * 2 + p

                @pl.when(ch + 1 < nchunk)
                def _():
                    _stage(ch + 1, 1 - p)

                _wait_stage(ch, p)
                _gather(p, 0, 0)

                @pl.loop(0, CH // 2)
                def _(t):
                    for b in range(2):
                        r = t * 2 + b
                        nb = 1 - b

                        @pl.when(r + 1 < CH)
                        def _():
                            @pl.when(r >= 1)
                            def _():
                                _wait_scatter(p, r - 1, nb)

                            _gather(p, r + 1, nb)

                        _wait_gather(p, r, b)

                        @pl.loop(0, 128)
                        def _(e):
                            cvec = plsc.load_gather(
                                coefs_v[p],
                                [jnp.full((LANES,), r * 128 + e, jnp.int32)])
                            for j in range(128 // LANES):
                                sl = pl.ds(j * LANES, LANES)
                                bufs[b][e, sl] = bufs[b][e, sl] * cvec

                        _scatter(p, r, b)

                _wait_scatter(p, CH - 2, 0)
                _wait_scatter(p, CH - 1, 1)

    with jax.named_scope("edge_phase"):
        @pl.when(c == 0)
        def _():
            _run(RPT0, s * RPT0)

        @pl.when(c == 1)
        def _():
            _run(RPT1, NR0TOT + s * RPT1)

        plsc.subcore_barrier()

    with jax.named_scope("copyout_phase"):
        for k in range(nz // 128):
            sl = pl.ds(s * nz + k * 128, 128)
            pltpu.sync_copy(acc_sh.at[sl], bufs[0])
            pltpu.sync_copy(bufs[0], out_hbm.at[c].at[sl])


@jax.jit
def _sc_gcn(h, src2d, dst2d, coef):
    return pl.kernel(
        _sc_gcn_body,
        out_type=jax.ShapeDtypeStruct((NC, NPAD, D), jnp.float32),
        mesh=_mesh,
        scratch_types=[
            [pltpu.VMEM((CH, 128), jnp.int32) for _ in range(2)],
            [pltpu.VMEM((CH, 128), jnp.int32) for _ in range(2)],
            [pltpu.VMEM((CH * 128,), jnp.float32) for _ in range(2)],
            [pltpu.VMEM((128, D), jnp.float32) for _ in range(2)],
            [pltpu.SemaphoreType.DMA for _ in range(2)],
            [pltpu.SemaphoreType.DMA for _ in range(2)],
            [pltpu.SemaphoreType.DMA for _ in range(2)],
            pltpu.VMEM_SHARED((NPAD, D), jnp.float32),
        ],
        compiler_params=_sc_params,
    )(h, src2d, dst2d, coef)


def _tc_inv_body(part_ref, inv_ref):
    deg = jnp.sum(part_ref[...], axis=(0, 1))
    inv_ref[...] = jnp.where(deg > 0.0,
                             lax.rsqrt(jnp.maximum(deg, 1e-12)), 0.0)


@jax.jit
def _tc_inv(part):
    return pl.pallas_call(
        _tc_inv_body,
        out_shape=jax.ShapeDtypeStruct((2, NPAD), jnp.float32),
    )(part)


_BLK = 2000


def _tc_pre_body(x_ref, w1_ref, wr_ref, br_ref, h1_ref, res_ref):
    x = x_ref[...]
    h1_ref[...] = jnp.dot(x, w1_ref[...], preferred_element_type=jnp.float32)
    res_ref[...] = (jnp.dot(x, wr_ref[...], preferred_element_type=jnp.float32)
                    + br_ref[...])


@jax.jit
def _tc_pre(x, W1, W_res, b_res):
    return pl.pallas_call(
        _tc_pre_body,
        grid=(N // _BLK,),
        in_specs=[
            pl.BlockSpec((_BLK, D), lambda i: (i, 0)),
            pl.BlockSpec((D, D), lambda i: (0, 0)),
            pl.BlockSpec((D, D), lambda i: (0, 0)),
            pl.BlockSpec((1, D), lambda i: (0, 0)),
        ],
        out_specs=[
            pl.BlockSpec((_BLK, D), lambda i: (i, 0)),
            pl.BlockSpec((_BLK, D), lambda i: (i, 0)),
        ],
        out_shape=[
            jax.ShapeDtypeStruct((N, D), jnp.float32),
            jax.ShapeDtypeStruct((N, D), jnp.float32),
        ],
    )(x, W1, W_res, b_res.reshape(1, D))


def _tc_mid_body(agg_ref, b_ref, w_ref, out_ref):
    a = agg_ref[0] + agg_ref[1] + b_ref[...]
    a = jnp.maximum(a, 0.0)
    out_ref[...] = jnp.dot(a, w_ref[...], preferred_element_type=jnp.float32)


@jax.jit
def _tc_mid(aggp, b, W_next):
    return pl.pallas_call(
        _tc_mid_body,
        grid=(N // _BLK,),
        in_specs=[
            pl.BlockSpec((2, _BLK, D), lambda i: (0, i, 0)),
            pl.BlockSpec((1, D), lambda i: (0, 0)),
            pl.BlockSpec((D, D), lambda i: (0, 0)),
        ],
        out_specs=pl.BlockSpec((_BLK, D), lambda i: (i, 0)),
        out_shape=jax.ShapeDtypeStruct((N, D), jnp.float32),
    )(aggp, b.reshape(1, D), W_next)


def _tc_fin_body(agg_ref, b4_ref, res_ref, wo_ref, bo_ref, out_ref):
    a = agg_ref[0] + agg_ref[1] + b4_ref[...] + res_ref[...]
    a = jnp.maximum(a, 0.0)
    out_ref[...] = (jnp.dot(a, wo_ref[...], preferred_element_type=jnp.float32)
                    + bo_ref[...])


@jax.jit
def _tc_fin(aggp, b4, res, W_op, b_op):
    C = W_op.shape[1]
    return pl.pallas_call(
        _tc_fin_body,
        grid=(N // _BLK,),
        in_specs=[
            pl.BlockSpec((2, _BLK, D), lambda i: (0, i, 0)),
            pl.BlockSpec((1, D), lambda i: (0, 0)),
            pl.BlockSpec((_BLK, D), lambda i: (i, 0)),
            pl.BlockSpec((D, C), lambda i: (0, 0)),
            pl.BlockSpec((1, C), lambda i: (0, 0)),
        ],
        out_specs=pl.BlockSpec((_BLK, C), lambda i: (i, 0)),
        out_shape=jax.ShapeDtypeStruct((N, C), jnp.float32),
    )(aggp, b4.reshape(1, D), res, W_op, b_op.reshape(1, C))


def kernel(x, edge_index, edge_weight, W_res, b_res, W1, b1, W2, b2, W3, b3,
           W4, b4, W_op, b_op):
    pad = EP - E
    src2d = jnp.pad(edge_index[0], (0, pad)).reshape(NROWS, 128)
    dst2d = jnp.pad(edge_index[1], (0, pad)).reshape(NROWS, 128)
    ew = jnp.pad(edge_weight, (0, pad))

    part = _sc_deg(src2d, dst2d, ew)
    inv = _tc_inv(part)
    coef = _sc_coef(inv, src2d, dst2d, ew)

    h1, res = _tc_pre(x, W1, W_res, b_res)

    agg1 = _sc_gcn(h1, src2d, dst2d, coef)
    h2 = _tc_mid(agg1, b1, W2)
    agg2 = _sc_gcn(h2, src2d, dst2d, coef)
    h3 = _tc_mid(agg2, b2, W3)
    agg3 = _sc_gcn(h3, src2d, dst2d, coef)
    h4 = _tc_mid(agg3, b3, W4)
    agg4 = _sc_gcn(h4, src2d, dst2d, coef)
    return _tc_fin(agg4, b4, res, W_op, b_op)


# speedup vs baseline: 1.0383x; 1.0383x over previous
import functools

import jax
import jax.numpy as jnp
from jax import lax
from jax.experimental import pallas as pl
from jax.experimental.pallas import tpu as pltpu
from jax.experimental.pallas import tpu_sc as plsc

N = 10000
E = 320000
D = 128
NC = 2
NS = 16
NW = NC * NS
LANES = 16

NPAD = 10240
EP = 327680
ROWS_PER_TILE = 80
EDGES_PER_TILE = ROWS_PER_TILE * 128
NROWS = EP // 128

_mesh = plsc.VectorSubcoreMesh(core_axis_name="c", subcore_axis_name="s")
_sc_params = pltpu.CompilerParams(needs_layout_passes=False)


def _wid():
    return lax.axis_index("c") * NS + lax.axis_index("s")


def _sc_deg_body(src_hbm, dst_hbm, ew_hbm, out_hbm,
                 src_v, dst_v, ew_v, degs_v, degd_v):
    w = _wid()
    c = lax.axis_index("c")
    s = lax.axis_index("s")
    row0 = w * ROWS_PER_TILE
    e0 = w * EDGES_PER_TILE

    @pl.loop(0, NPAD // LANES)
    def _(k):
        z = jnp.zeros((LANES,), jnp.float32)
        degs_v[pl.ds(k * LANES, LANES)] = z
        degd_v[pl.ds(k * LANES, LANES)] = z

    pltpu.sync_copy(src_hbm.at[pl.ds(row0, ROWS_PER_TILE)], src_v)
    pltpu.sync_copy(dst_hbm.at[pl.ds(row0, ROWS_PER_TILE)], dst_v)
    pltpu.sync_copy(ew_hbm.at[pl.ds(e0, EDGES_PER_TILE)], ew_v)

    @pl.loop(0, ROWS_PER_TILE)
    def _(i):
        for j in range(128 // LANES):
            sl = pl.ds(j * LANES, LANES)
            ev = ew_v[pl.ds(i * 128 + j * LANES, LANES)]
            plsc.addupdate_scatter(degs_v, [src_v[i, sl]], ev)
            plsc.addupdate_scatter(degd_v, [dst_v[i, sl]], ev)

    pltpu.sync_copy(degs_v, out_hbm.at[c, s, 0])
    pltpu.sync_copy(degd_v, out_hbm.at[c, s, 1])


@jax.jit
def _sc_deg(src2d, dst2d, ew):
    return pl.kernel(
        _sc_deg_body,
        out_type=jax.ShapeDtypeStruct((NC, NS, 2, NPAD), jnp.float32),
        mesh=_mesh,
        scratch_types=[
            pltpu.VMEM((ROWS_PER_TILE, 128), jnp.int32),
            pltpu.VMEM((ROWS_PER_TILE, 128), jnp.int32),
            pltpu.VMEM((EDGES_PER_TILE,), jnp.float32),
            pltpu.VMEM((NPAD,), jnp.float32),
            pltpu.VMEM((NPAD,), jnp.float32),
        ],
        compiler_params=_sc_params,
    )(src2d, dst2d, ew)


def _sc_coef_body(inv_hbm, src_hbm, dst_hbm, ew_hbm, coef_hbm,
                  invs_v, invd_v, src_v, dst_v, ew_v, coef_v):
    w = _wid()
    row0 = w * ROWS_PER_TILE
    e0 = w * EDGES_PER_TILE

    pltpu.sync_copy(inv_hbm.at[0], invs_v)
    pltpu.sync_copy(inv_hbm.at[1], invd_v)
    pltpu.sync_copy(src_hbm.at[pl.ds(row0, ROWS_PER_TILE)], src_v)
    pltpu.sync_copy(dst_hbm.at[pl.ds(row0, ROWS_PER_TILE)], dst_v)
    pltpu.sync_copy(ew_hbm.at[pl.ds(e0, EDGES_PER_TILE)], ew_v)

    @pl.loop(0, ROWS_PER_TILE)
    def _(i):
        for j in range(128 // LANES):
            sl = pl.ds(j * LANES, LANES)
            fl = pl.ds(i * 128 + j * LANES, LANES)
            a = plsc.load_gather(invs_v, [src_v[i, sl]])
            b = plsc.load_gather(invd_v, [dst_v[i, sl]])
            coef_v[fl] = ew_v[fl] * a * b

    pltpu.sync_copy(coef_v, coef_hbm.at[pl.ds(e0, EDGES_PER_TILE)])


@jax.jit
def _sc_coef(inv, src2d, dst2d, ew):
    return pl.kernel(
        _sc_coef_body,
        out_type=jax.ShapeDtypeStruct((EP,), jnp.float32),
        mesh=_mesh,
        scratch_types=[
            pltpu.VMEM((NPAD,), jnp.float32),
            pltpu.VMEM((NPAD,), jnp.float32),
            pltpu.VMEM((ROWS_PER_TILE, 128), jnp.int32),
            pltpu.VMEM((ROWS_PER_TILE, 128), jnp.int32),
            pltpu.VMEM((EDGES_PER_TILE,), jnp.float32),
            pltpu.VMEM((EDGES_PER_TILE,), jnp.float32),
        ],
        compiler_params=_sc_params,
    )(inv, src2d, dst2d, ew)


CH = 8
GSPLIT = 2
NCHUNK = ROWS_PER_TILE // CH


RPT0 = 144
RPT1 = 16
NR0TOT = NS * RPT0
assert RPT0 + RPT1 == 2 * ROWS_PER_TILE
assert RPT0 % (2 * CH) == 0 and RPT1 % (2 * CH) == 0


def _sc_gcn_body(h_hbm, src_hbm, dst_hbm, coef_hbm, out_hbm,
                 srcs_v, dsts_v, coefs_v, bufs, gsems, ssems, stgsems,
                 acc_sh):
    c = lax.axis_index("c")
    s = lax.axis_index("s")
    nz = NPAD // NS

    with jax.named_scope("zero_phase"):
        @pl.loop(0, 128)
        def _(i):
            for j in range(128 // LANES):
                bufs[0][i, pl.ds(j * LANES, LANES)] = jnp.zeros(
                    (LANES,), jnp.float32)

        for k in range(nz // 128):
            pltpu.sync_copy(bufs[0], acc_sh.at[pl.ds(s * nz + k * 128, 128)])
        plsc.subcore_barrier()

    def _run(rpt, row0):
        e0 = row0 * 128
        nchunk = rpt // CH

        def _stage(ch, p):
            pltpu.async_copy(src_hbm.at[pl.ds(row0 + ch * CH, CH)],
                             srcs_v[p], stgsems[p])
            pltpu.async_copy(dst_hbm.at[pl.ds(row0 + ch * CH, CH)],
                             dsts_v[p], stgsems[p])
            pltpu.async_copy(coef_hbm.at[pl.ds(e0 + ch * CH * 128, CH * 128)],
                             coefs_v[p], stgsems[p])

        def _wait_stage(ch, p):
            pltpu.make_async_copy(src_hbm.at[pl.ds(row0 + ch * CH, CH)],
                                  srcs_v[p], stgsems[p]).wait()
            pltpu.make_async_copy(dst_hbm.at[pl.ds(row0 + ch * CH, CH)],
                                  dsts_v[p], stgsems[p]).wait()
            pltpu.make_async_copy(
                coef_hbm.at[pl.ds(e0 + ch * CH * 128, CH * 128)],
                coefs_v[p], stgsems[p]).wait()

        def _gather(p, r, b):
            for h in range(GSPLIT):
                hs = pl.ds(h * (128 // GSPLIT), 128 // GSPLIT)
                pltpu.async_copy(h_hbm.at[srcs_v[p].at[r, hs]],
                                 bufs[b].at[hs], gsems[b])

        def _wait_gather(p, r, b):
            for h in range(GSPLIT):
                hs = pl.ds(h * (128 // GSPLIT), 128 // GSPLIT)
                pltpu.make_async_copy(h_hbm.at[srcs_v[p].at[r, hs]],
                                      bufs[b].at[hs], gsems[b]).wait()

        def _scatter(p, r, b):
            pltpu.async_copy(bufs[b], acc_sh.at[dsts_v[p].at[r]], ssems[b],
                             add=True)

        def _wait_scatter(p, r, b):
            pltpu.make_async_copy(bufs[b], acc_sh.at[dsts_v[p].at[r]],
                                  ssems[b]).wait()

        _stage(0, 0)

        @pl.loop(0, nchunk // 2)
        def _(g):
            for p in range(2):
                ch = g * 2 + p

                @pl.when(ch + 1 < nchunk)
                def _():
                    _stage(ch + 1, 1 - p)

                _wait_stage(ch, p)
                _gather(p, 0, 0)

                @pl.loop(0, CH // 2)
                def _(t):
                    for b in range(2):
                        r = t * 2 + b
                        nb = 1 - b

                        @pl.when(r + 1 < CH)
                        def _():
                            @pl.when(r >= 1)
                            def _():
                                _wait_scatter(p, r - 1, nb)

                            _gather(p, r + 1, nb)

                        _wait_gather(p, r, b)

                        @pl.loop(0, 128)
                        def _(e):
                            cvec = plsc.load_gather(
                                coefs_v[p],
                                [jnp.full((LANES,), r * 128 + e, jnp.int32)])
                            for j in range(128 // LANES):
                                sl = pl.ds(j * LANES, LANES)
                                bufs[b][e, sl] = bufs[b][e, sl] * cvec

                        _scatter(p, r, b)

                _wait_scatter(p, CH - 2, 0)
                _wait_scatter(p, CH - 1, 1)

    with jax.named_scope("edge_phase"):
        @pl.when(c == 0)
        def _():
            _run(RPT0, s * RPT0)

        @pl.when(c == 1)
        def _():
            _run(RPT1, NR0TOT + s * RPT1)

        plsc.subcore_barrier()

    with jax.named_scope("copyout_phase"):
        for k in range(nz // 128):
            sl = pl.ds(s * nz + k * 128, 128)
            pltpu.sync_copy(acc_sh.at[sl], bufs[0])
            pltpu.sync_copy(bufs[0], out_hbm.at[c].at[sl])


@jax.jit
def _sc_gcn(h, src2d, dst2d, coef):
    return pl.kernel(
        _sc_gcn_body,
        out_type=jax.ShapeDtypeStruct((NC, NPAD, D), jnp.float32),
        mesh=_mesh,
        scratch_types=[
            [pltpu.VMEM((CH, 128), jnp.int32) for _ in range(2)],
            [pltpu.VMEM((CH, 128), jnp.int32) for _ in range(2)],
            [pltpu.VMEM((CH * 128,), jnp.float32) for _ in range(2)],
            [pltpu.VMEM((128, D), jnp.float32) for _ in range(2)],
            [pltpu.SemaphoreType.DMA for _ in range(2)],
            [pltpu.SemaphoreType.DMA for _ in range(2)],
            [pltpu.SemaphoreType.DMA for _ in range(2)],
            pltpu.VMEM_SHARED((NPAD, D), jnp.float32),
        ],
        compiler_params=_sc_params,
    )(h, src2d, dst2d, coef)


def _tc_inv_body(part_ref, inv_ref):
    deg = jnp.sum(part_ref[...], axis=(0, 1))
    inv_ref[...] = jnp.where(deg > 0.0,
                             lax.rsqrt(jnp.maximum(deg, 1e-12)), 0.0)


@jax.jit
def _tc_inv(part):
    return pl.pallas_call(
        _tc_inv_body,
        out_shape=jax.ShapeDtypeStruct((2, NPAD), jnp.float32),
    )(part)


_BLK = 2000


def _tc_pre_body(x_ref, w1_ref, wr_ref, br_ref, h1_ref, res_ref):
    x = x_ref[...]
    h1_ref[...] = jnp.dot(x, w1_ref[...], preferred_element_type=jnp.float32)
    res_ref[...] = (jnp.dot(x, wr_ref[...], preferred_element_type=jnp.float32)
                    + br_ref[...])


@jax.jit
def _tc_pre(x, W1, W_res, b_res):
    return pl.pallas_call(
        _tc_pre_body,
        grid=(N // _BLK,),
        in_specs=[
            pl.BlockSpec((_BLK, D), lambda i: (i, 0)),
            pl.BlockSpec((D, D), lambda i: (0, 0)),
            pl.BlockSpec((D, D), lambda i: (0, 0)),
            pl.BlockSpec((1, D), lambda i: (0, 0)),
        ],
        out_specs=[
            pl.BlockSpec((_BLK, D), lambda i: (i, 0)),
            pl.BlockSpec((_BLK, D), lambda i: (i, 0)),
        ],
        out_shape=[
            jax.ShapeDtypeStruct((N, D), jnp.float32),
            jax.ShapeDtypeStruct((N, D), jnp.float32),
        ],
    )(x, W1, W_res, b_res.reshape(1, D))


def _tc_mid_body(agg_ref, b_ref, w_ref, out_ref):
    a = agg_ref[0] + agg_ref[1] + b_ref[...]
    a = jnp.maximum(a, 0.0)
    out_ref[...] = jnp.dot(a, w_ref[...], preferred_element_type=jnp.float32)


@jax.jit
def _tc_mid(aggp, b, W_next):
    return pl.pallas_call(
        _tc_mid_body,
        grid=(N // _BLK,),
        in_specs=[
            pl.BlockSpec((2, _BLK, D), lambda i: (0, i, 0)),
            pl.BlockSpec((1, D), lambda i: (0, 0)),
            pl.BlockSpec((D, D), lambda i: (0, 0)),
        ],
        out_specs=pl.BlockSpec((_BLK, D), lambda i: (i, 0)),
        out_shape=jax.ShapeDtypeStruct((N, D), jnp.float32),
    )(aggp, b.reshape(1, D), W_next)


def _tc_fin_body(agg_ref, b4_ref, res_ref, wo_ref, bo_ref, out_ref):
    a = agg_ref[0] + agg_ref[1] + b4_ref[...] + res_ref[...]
    a = jnp.maximum(a, 0.0)
    out_ref[...] = (jnp.dot(a, wo_ref[...], preferred_element_type=jnp.float32)
                    + bo_ref[...])


@jax.jit
def _tc_fin(aggp, b4, res, W_op, b_op):
    C = W_op.shape[1]
    return pl.pallas_call(
        _tc_fin_body,
        grid=(N // _BLK,),
        in_specs=[
            pl.BlockSpec((2, _BLK, D), lambda i: (0, i, 0)),
            pl.BlockSpec((1, D), lambda i: (0, 0)),
            pl.BlockSpec((_BLK, D), lambda i: (i, 0)),
            pl.BlockSpec((D, C), lambda i: (0, 0)),
            pl.BlockSpec((1, C), lambda i: (0, 0)),
        ],
        out_specs=pl.BlockSpec((_BLK, C), lambda i: (i, 0)),
        out_shape=jax.ShapeDtypeStruct((N, C), jnp.float32),
    )(aggp, b4.reshape(1, D), res, W_op, b_op.reshape(1, C))


def kernel(x, edge_index, edge_weight, W_res, b_res, W1, b1, W2, b2, W3, b3,
           W4, b4, W_op, b_op):
    pad = EP - E
    src2d = jnp.pad(edge_index[0], (0, pad)).reshape(NROWS, 128)
    dst2d = jnp.pad(edge_index[1], (0, pad)).reshape(NROWS, 128)
    ew = jnp.pad(edge_weight, (0, pad))

    part = _sc_deg(src2d, dst2d, ew)
    inv = _tc_inv(part)
    coef = _sc_coef(inv, src2d, dst2d, ew)

    h1, res = _tc_pre(x, W1, W_res, b_res)

    agg1 = _sc_gcn(h1, src2d, dst2d, coef)
    h2 = _tc_mid(agg1, b1, W2)
    agg2 = _sc_gcn(h2, src2d, dst2d, coef)
    h3 = _tc_mid(agg2, b2, W3)
    agg3 = _sc_gcn(h3, src2d, dst2d, coef)
    h4 = _tc_mid(agg3, b3, W4)
    agg4 = _sc_gcn(h4, src2d, dst2d, coef)
    return _tc_fin(agg4, b4, res, W_op, b_op)
